# baseline (speedup 1.0000x reference)
import functools

import jax
import jax.numpy as jnp
from jax import lax
from jax.experimental import pallas as pl
from jax.experimental.pallas import tpu as pltpu
from jax.experimental.pallas import tpu_sc as plsc

B, S, D = 4, 4096, 1024
NC, NS = 2, 16
NW = NC * NS
SEQ_PER_W = S // NW
ROWS = 8
HD = D // 2
N_UNITS = (SEQ_PER_W // ROWS) * 2
NSLOT = 4
N_STEPS = N_UNITS // NSLOT
LANES = 16
HCOLS = HD // LANES
UNROLL = 2

_mesh = plsc.VectorSubcoreMesh(core_axis_name="c", subcore_axis_name="s")


@functools.partial(
    pl.kernel,
    out_type=jax.ShapeDtypeStruct((B, S, D), jnp.float32),
    mesh=_mesh,
    scratch_types=[
        [pltpu.VMEM((B, ROWS, HD), jnp.float32) for _ in range(NSLOT)],
        [pltpu.VMEM((ROWS, HD), jnp.float32) for _ in range(2)],
        [pltpu.SemaphoreType.DMA for _ in range(NSLOT)],
        [pltpu.SemaphoreType.DMA for _ in range(NSLOT)],
        [pltpu.SemaphoreType.DMA for _ in range(2)],
    ],
)
def _pe_add(te_hbm, enc_hbm, out_hbm, slots, ebufs, in_sems, out_sems, e_sems):
    wid = lax.axis_index("s") * NC + lax.axis_index("c")
    s_base = wid * SEQ_PER_W

    def unit_origin(u):
        s0 = pl.multiple_of(jnp.clip(s_base + (u >> 1) * ROWS, 0, S - ROWS), ROWS)
        d0 = pl.multiple_of((u & 1) * HD, HD)
        return s0, d0

    def in_copy(u, j):
        s0, d0 = unit_origin(u)
        return pltpu.make_async_copy(
            te_hbm.at[:, pl.ds(s0, ROWS), pl.ds(d0, HD)], slots[j], in_sems[j]
        )

    def out_copy(u, j):
        s0, d0 = unit_origin(u)
        return pltpu.make_async_copy(
            slots[j], out_hbm.at[:, pl.ds(s0, ROWS), pl.ds(d0, HD)], out_sems[j]
        )

    def enc_copy(u, p):
        s0, d0 = unit_origin(u)
        return pltpu.make_async_copy(
            enc_hbm.at[pl.ds(s0, ROWS), pl.ds(d0, HD)], ebufs[p], e_sems[p]
        )

    in_copy(0, 0).start()
    in_copy(1, 1).start()
    enc_copy(0, 0).start()

    def step(k, carry):
        for j in range(NSLOT):
            u = k * NSLOT + j
            enc_copy(u + 1, (j + 1) % 2).start()
            in_copy(u, j).wait()
            enc_copy(u, j % 2).wait()

            tbuf = slots[j]
            ebuf = ebufs[j % 2]

            @plsc.parallel_loop(0, HCOLS, step=1, unroll=UNROLL)
            def body(i):
                col = i * LANES
                for row in range(ROWS):
                    e = ebuf[row, pl.ds(col, LANES)]
                    for b in range(B):
                        tbuf[b, row, pl.ds(col, LANES)] = (
                            tbuf[b, row, pl.ds(col, LANES)] + e
                        )

            jo = (j + 2) % NSLOT

            @pl.when(u >= 2)
            def _():
                out_copy(u - 2, jo).wait()

            in_copy(u + 2, jo).start()
            out_copy(u, j).start()
        return carry

    lax.fori_loop(0, N_STEPS, step, 0)

    out_copy(N_UNITS - 2, 2).wait()
    out_copy(N_UNITS - 1, 3).wait()
    in_copy(N_UNITS, 0).wait()
    in_copy(N_UNITS + 1, 1).wait()
    enc_copy(N_UNITS, 0).wait()


def kernel(token_embeddings, encoding):
    return _pe_add(token_embeddings, encoding)


# speedup vs baseline: 1.1035x; 1.1035x over previous
import functools

import jax
import jax.numpy as jnp
from jax import lax
from jax.experimental import pallas as pl
from jax.experimental.pallas import tpu as pltpu
from jax.experimental.pallas import tpu_sc as plsc

B, S, D = 4, 4096, 1024
NC, NS = 2, 16
NW = NC * NS
SEQ_PER_W = S // NW
ROWS = 8
HD = D // 2
N_UNITS = (SEQ_PER_W // ROWS) * 2
NSLOT = 4
N_STEPS = N_UNITS // NSLOT
LANES = 16
HCOLS = HD // LANES
UNROLL = 1

_mesh = plsc.VectorSubcoreMesh(core_axis_name="c", subcore_axis_name="s")


@functools.partial(
    pl.kernel,
    out_type=jax.ShapeDtypeStruct((B, S, D), jnp.float32),
    mesh=_mesh,
    scratch_types=[
        [pltpu.VMEM((B, ROWS, HD), jnp.float32) for _ in range(NSLOT)],
        [pltpu.VMEM((ROWS, HD), jnp.float32) for _ in range(2)],
        [pltpu.SemaphoreType.DMA for _ in range(NSLOT)],
        [pltpu.SemaphoreType.DMA for _ in range(NSLOT)],
        [pltpu.SemaphoreType.DMA for _ in range(2)],
    ],
)
def _pe_add(te_hbm, enc_hbm, out_hbm, slots, ebufs, in_sems, out_sems, e_sems):
    wid = lax.axis_index("s") * NC + lax.axis_index("c")
    s_base = wid * SEQ_PER_W

    def unit_origin(u):
        s0 = pl.multiple_of(jnp.clip(s_base + (u >> 1) * ROWS, 0, S - ROWS), ROWS)
        d0 = pl.multiple_of((u & 1) * HD, HD)
        return s0, d0

    def in_copy(u, j):
        s0, d0 = unit_origin(u)
        return pltpu.make_async_copy(
            te_hbm.at[:, pl.ds(s0, ROWS), pl.ds(d0, HD)], slots[j], in_sems[j]
        )

    def out_copy(u, j):
        s0, d0 = unit_origin(u)
        return pltpu.make_async_copy(
            slots[j], out_hbm.at[:, pl.ds(s0, ROWS), pl.ds(d0, HD)], out_sems[j]
        )

    def enc_copy(u, p):
        s0, d0 = unit_origin(u)
        return pltpu.make_async_copy(
            enc_hbm.at[pl.ds(s0, ROWS), pl.ds(d0, HD)], ebufs[p], e_sems[p]
        )

    in_copy(0, 0).start()
    in_copy(1, 1).start()
    enc_copy(0, 0).start()

    def step(k, carry):
        for j in range(NSLOT):
            u = k * NSLOT + j
            enc_copy(u + 1, (j + 1) % 2).start()
            in_copy(u, j).wait()
            enc_copy(u, j % 2).wait()

            tbuf = slots[j]
            ebuf = ebufs[j % 2]

            @plsc.parallel_loop(0, HCOLS, step=1, unroll=UNROLL)
            def body(i):
                col = i * LANES
                for row in range(ROWS):
                    e = ebuf[row, pl.ds(col, LANES)]
                    for b in range(B):
                        tbuf[b, row, pl.ds(col, LANES)] = (
                            tbuf[b, row, pl.ds(col, LANES)] + e
                        )

            jo = (j + 2) % NSLOT

            @pl.when(u >= 2)
            def _():
                out_copy(u - 2, jo).wait()

            in_copy(u + 2, jo).start()
            out_copy(u, j).start()
        return carry

    lax.fori_loop(0, N_STEPS, step, 0)

    out_copy(N_UNITS - 2, 2).wait()
    out_copy(N_UNITS - 1, 3).wait()
    in_copy(N_UNITS, 0).wait()
    in_copy(N_UNITS + 1, 1).wait()
    enc_copy(N_UNITS, 0).wait()


def kernel(token_embeddings, encoding):
    return _pe_add(token_embeddings, encoding)
